# baseline (speedup 1.0000x reference)
import jax
import jax.numpy as jnp
from jax import lax
from jax.experimental import pallas as pl
from jax.experimental.pallas import tpu as pltpu
from jax.experimental.pallas import tpu_sc as plsc

P_DROP = 0.1
N_FRAMES = 512
TOTAL = 512 * 3 * 224 * 224
TC_HB = 16
ROWBLK = TC_HB * 224 * 512
NTCBLK = 24
SC_OFF = NTCBLK * ROWBLK
SC_TOTAL = TOTAL - SC_OFF
NC, NS = 2, 16
NW = NC * NS
SHARD = SC_TOTAL // NW
BLK = 4096
CHUNK = 7 * BLK
NCHUNK = SHARD // CHUNK
NB = 4
NSTEP = NCHUNK // NB


def _body(img_hbm, rand_hbm, out_hbm, rv, *rest):
    bufs = rest[:NB]
    gsems = rest[NB:2 * NB]
    ssems = rest[2 * NB:3 * NB]

    wid = lax.axis_index("s") * NC + lax.axis_index("c")
    w0 = SC_OFF + wid * SHARD

    pltpu.sync_copy(rand_hbm, rv)
    mregs = []
    for g in range(32):
        v = rv[pl.ds(g * 16, 16)]
        mregs.append(jnp.where(v >= P_DROP, jnp.float32(1.0),
                               jnp.float32(0.0)))

    def _vpass(buf):
        def _block(blk, carry):
            base = blk * BLK
            for tn in range(4):
                for lg in range(8):
                    m = mregs[tn * 8 + lg]
                    for sub in range(8):
                        off = base + tn * 1024 + sub * 128 + lg * 16
                        buf[pl.ds(off, 16)] = buf[pl.ds(off, 16)] * m
            return carry
        lax.fori_loop(0, CHUNK // BLK, _block, 0)

    def _chunk(step, b):
        return w0 + (step * NB + b) * CHUNK

    for b in range(NB):
        pltpu.async_copy(img_hbm.at[pl.ds(_chunk(0, b), CHUNK)],
                         bufs[b], gsems[b])

    def _step(step, carry):
        for b in range(NB):
            pltpu.make_async_copy(
                img_hbm.at[pl.ds(_chunk(step, b), CHUNK)],
                bufs[b], gsems[b]).wait()
            _vpass(bufs[b])
            pltpu.async_copy(bufs[b], out_hbm.at[pl.ds(_chunk(step, b), CHUNK)],
                             ssems[b])
        for b in range(NB):
            pltpu.make_async_copy(
                bufs[b], out_hbm.at[pl.ds(_chunk(step, b), CHUNK)],
                ssems[b]).wait()

            @pl.when(step + 1 < NSTEP)
            def _next():
                pltpu.async_copy(img_hbm.at[pl.ds(_chunk(step + 1, b), CHUNK)],
                                 bufs[b], gsems[b])
        return carry

    lax.fori_loop(0, NSTEP, _step, 0)


def _drop_frames_sc(img_flat, rand_nums):
    mesh = plsc.VectorSubcoreMesh(core_axis_name="c", subcore_axis_name="s")
    run = pl.kernel(
        _body,
        mesh=mesh,
        out_type=jax.ShapeDtypeStruct((TOTAL,), jnp.float32),
        scratch_types=[
            pltpu.VMEM((N_FRAMES,), jnp.float32),
        ] + [pltpu.VMEM((CHUNK,), jnp.float32)] * NB
          + [pltpu.SemaphoreType.DMA] * (2 * NB),
    )
    return run(img_flat, rand_nums)


def _tc_body(rand_ref, alias_ref, x_ref, o_ref):
    del alias_ref
    keep = (rand_ref[...] >= P_DROP).astype(jnp.float32)
    o_ref[...] = x_ref[...] * keep[None, None, None, :]


def _drop_frames_tc(sc4, x4, rand_nums):
    nhb = 224 // TC_HB
    return pl.pallas_call(
        _tc_body,
        grid=(NTCBLK,),
        in_specs=[
            pl.BlockSpec((N_FRAMES,), lambda r: (0,)),
            pl.BlockSpec(memory_space=pl.ANY),
            pl.BlockSpec((1, TC_HB, 224, 512),
                         lambda r: (r // nhb, r % nhb, 0, 0)),
        ],
        out_specs=pl.BlockSpec((1, TC_HB, 224, 512),
                               lambda r: (r // nhb, r % nhb, 0, 0)),
        out_shape=jax.ShapeDtypeStruct((3, 224, 224, 512), jnp.float32),
        input_output_aliases={1: 0},
    )(rand_nums, sc4, x4)


def kernel(img, rand_nums):
    x = jnp.transpose(img, (1, 2, 3, 0))
    xp = x.reshape(3, 224, 28, 8, 4, 128)
    xp = jnp.transpose(xp, (0, 1, 2, 4, 3, 5))
    flat = xp.reshape(-1)

    sc_out = _drop_frames_sc(flat, rand_nums)
    y = sc_out.reshape(3, 224, 28, 4, 8, 128)
    y = jnp.transpose(y, (0, 1, 2, 4, 3, 5))
    y = y.reshape(3, 224, 224, 512)

    full = _drop_frames_tc(y, x, rand_nums)
    return jnp.transpose(full, (3, 0, 1, 2))


# speedup vs baseline: 1.0110x; 1.0110x over previous
import jax
import jax.numpy as jnp
from jax import lax
from jax.experimental import pallas as pl
from jax.experimental.pallas import tpu as pltpu
from jax.experimental.pallas import tpu_sc as plsc

P_DROP = 0.1
N_FRAMES = 512
TOTAL = 512 * 3 * 224 * 224
TC_HB = 16
ROWBLK = TC_HB * 224 * 512
NTCBLK = 21
SC_OFF = NTCBLK * ROWBLK
SC_TOTAL = TOTAL - SC_OFF
NC, NS = 2, 16
NW = NC * NS
SHARD = SC_TOTAL // NW
BLK = 4096
CHUNK = 7 * BLK
NCHUNK = SHARD // CHUNK
NB = 3
NSTEP = NCHUNK // NB


def _body(img_hbm, rand_hbm, out_hbm, rv, *rest):
    bufs = rest[:NB]
    gsems = rest[NB:2 * NB]
    ssems = rest[2 * NB:3 * NB]

    wid = lax.axis_index("s") * NC + lax.axis_index("c")
    w0 = SC_OFF + wid * SHARD

    pltpu.sync_copy(rand_hbm, rv)
    mregs = []
    for g in range(32):
        v = rv[pl.ds(g * 16, 16)]
        mregs.append(jnp.where(v >= P_DROP, jnp.float32(1.0),
                               jnp.float32(0.0)))

    def _vpass(buf):
        def _block(blk, carry):
            base = blk * BLK
            for tn in range(4):
                for lg in range(8):
                    m = mregs[tn * 8 + lg]
                    for sub in range(8):
                        off = base + tn * 1024 + sub * 128 + lg * 16
                        buf[pl.ds(off, 16)] = buf[pl.ds(off, 16)] * m
            return carry
        lax.fori_loop(0, CHUNK // BLK, _block, 0)

    def _chunk(step, b):
        return w0 + (step * NB + b) * CHUNK

    for b in range(NB):
        pltpu.async_copy(img_hbm.at[pl.ds(_chunk(0, b), CHUNK)],
                         bufs[b], gsems[b])

    def _step(step, carry):
        for b in range(NB):
            pltpu.make_async_copy(
                img_hbm.at[pl.ds(_chunk(step, b), CHUNK)],
                bufs[b], gsems[b]).wait()
            _vpass(bufs[b])
            pltpu.async_copy(bufs[b], out_hbm.at[pl.ds(_chunk(step, b), CHUNK)],
                             ssems[b])
        for b in range(NB):
            pltpu.make_async_copy(
                bufs[b], out_hbm.at[pl.ds(_chunk(step, b), CHUNK)],
                ssems[b]).wait()

            @pl.when(step + 1 < NSTEP)
            def _next():
                pltpu.async_copy(img_hbm.at[pl.ds(_chunk(step + 1, b), CHUNK)],
                                 bufs[b], gsems[b])
        return carry

    lax.fori_loop(0, NSTEP, _step, 0)


def _drop_frames_sc(img_flat, rand_nums):
    mesh = plsc.VectorSubcoreMesh(core_axis_name="c", subcore_axis_name="s")
    run = pl.kernel(
        _body,
        mesh=mesh,
        out_type=jax.ShapeDtypeStruct((TOTAL,), jnp.float32),
        scratch_types=[
            pltpu.VMEM((N_FRAMES,), jnp.float32),
        ] + [pltpu.VMEM((CHUNK,), jnp.float32)] * NB
          + [pltpu.SemaphoreType.DMA] * (2 * NB),
    )
    return run(img_flat, rand_nums)


def _tc_body(rand_ref, alias_ref, x_ref, o_ref):
    del alias_ref
    keep = (rand_ref[...] >= P_DROP).astype(jnp.float32)
    o_ref[...] = x_ref[...] * keep[None, None, None, :]


def _drop_frames_tc(sc4, x4, rand_nums):
    nhb = 224 // TC_HB
    return pl.pallas_call(
        _tc_body,
        grid=(NTCBLK,),
        in_specs=[
            pl.BlockSpec((N_FRAMES,), lambda r: (0,)),
            pl.BlockSpec(memory_space=pl.ANY),
            pl.BlockSpec((1, TC_HB, 224, 512),
                         lambda r: (r // nhb, r % nhb, 0, 0)),
        ],
        out_specs=pl.BlockSpec((1, TC_HB, 224, 512),
                               lambda r: (r // nhb, r % nhb, 0, 0)),
        out_shape=jax.ShapeDtypeStruct((3, 224, 224, 512), jnp.float32),
        input_output_aliases={1: 0},
    )(rand_nums, sc4, x4)


def kernel(img, rand_nums):
    x = jnp.transpose(img, (1, 2, 3, 0))
    xp = x.reshape(3, 224, 28, 8, 4, 128)
    xp = jnp.transpose(xp, (0, 1, 2, 4, 3, 5))
    flat = xp.reshape(-1)

    sc_out = _drop_frames_sc(flat, rand_nums)
    y = sc_out.reshape(3, 224, 28, 4, 8, 128)
    y = jnp.transpose(y, (0, 1, 2, 4, 3, 5))
    y = y.reshape(3, 224, 224, 512)

    full = _drop_frames_tc(y, x, rand_nums)
    return jnp.transpose(full, (3, 0, 1, 2))
